# baseline (speedup 1.0000x reference)
import jax
import jax.numpy as jnp
from jax.experimental import pallas as pl
from jax.experimental.pallas import tpu as pltpu


def _moe_body(x_ref, Wg_ref, bg_ref, Wa_ref, Wb_ref, be_ref, out_ref,
              probs_ref, hacc_ref):
    e = pl.program_id(0)
    k = pl.program_id(1)

    @pl.when((e == 0) & (k == 0))
    def _init():
        logits = (
            jnp.dot(x_ref[...], Wg_ref[...], preferred_element_type=jnp.float32)
            + bg_ref[...]
        )
        m = jnp.max(logits, axis=-1, keepdims=True)
        p = jnp.exp(logits - m)
        probs_ref[...] = p / jnp.sum(p, axis=-1, keepdims=True)

    kq = Wa_ref.shape[3]

    @pl.when(k == 0)
    def _khalf0():
        part = jnp.dot(x_ref[:, :kq], Wa_ref[0, 0, 0],
                       preferred_element_type=jnp.float32)
        part += jnp.dot(x_ref[:, kq:2 * kq], Wb_ref[0, 0, 0],
                        preferred_element_type=jnp.float32)
        hacc_ref[...] = part

    @pl.when(k == 1)
    def _khalf1():
        h = hacc_ref[...]
        h += jnp.dot(x_ref[:, 2 * kq:3 * kq], Wa_ref[0, 0, 0],
                     preferred_element_type=jnp.float32)
        h += jnp.dot(x_ref[:, 3 * kq:], Wb_ref[0, 0, 0],
                     preferred_element_type=jnp.float32)
        h = jnp.maximum(h + be_ref[0, 0], 0.0)

        lane = jax.lax.broadcasted_iota(jnp.int32, probs_ref.shape, 1)
        p_col = jnp.sum(
            jnp.where(lane == e, probs_ref[...], 0.0), axis=1, keepdims=True
        )
        contrib = h * p_col

        @pl.when(e == 0)
        def _first():
            out_ref[...] = contrib

        @pl.when(e > 0)
        def _rest():
            out_ref[...] += contrib


def kernel(x, Wg, bg, We, be):
    T, H = x.shape
    E = We.shape[0]
    bg2 = bg.reshape(1, E)
    be3 = be.reshape(E, 1, H)
    We5 = We.reshape(E, 2, 2, H // 4, H)
    return pl.pallas_call(
        _moe_body,
        grid=(E, 2),
        in_specs=[
            pl.BlockSpec((T, H), lambda e, k: (0, 0)),
            pl.BlockSpec((H, E), lambda e, k: (0, 0)),
            pl.BlockSpec((1, E), lambda e, k: (0, 0)),
            pl.BlockSpec((1, 1, 1, H // 4, H), lambda e, k: (e, k, 0, 0, 0)),
            pl.BlockSpec((1, 1, 1, H // 4, H), lambda e, k: (e, k, 1, 0, 0)),
            pl.BlockSpec((1, 1, H), lambda e, k: (e, 0, 0)),
        ],
        out_specs=pl.BlockSpec((T, H), lambda e, k: (0, 0)),
        out_shape=jax.ShapeDtypeStruct((T, H), jnp.float32),
        scratch_shapes=[
            pltpu.VMEM((T, E), jnp.float32),
            pltpu.VMEM((T, H), jnp.float32),
        ],
        compiler_params=pltpu.CompilerParams(
            dimension_semantics=("arbitrary", "arbitrary"),
        ),
    )(x, Wg, bg2, We5, We5, be3)


# speedup vs baseline: 1.1420x; 1.1420x over previous
import jax
import jax.numpy as jnp
from jax.experimental import pallas as pl
from jax.experimental.pallas import tpu as pltpu


def _moe_body(x_ref, Wg_ref, bg_ref, Wa_ref, Wb_ref, be_ref, out_ref, probs_ref):
    e = pl.program_id(0)

    @pl.when(e == 0)
    def _init():
        logits = (
            jnp.dot(x_ref[...], Wg_ref[...], preferred_element_type=jnp.float32)
            + bg_ref[...]
        )
        m = jnp.max(logits, axis=-1, keepdims=True)
        p = jnp.exp(logits - m)
        probs_ref[...] = p / jnp.sum(p, axis=-1, keepdims=True)

    kh = Wa_ref.shape[2]
    h = jnp.dot(x_ref[:, :kh], Wa_ref[0, 0], preferred_element_type=jnp.float32)
    h += jnp.dot(x_ref[:, kh:], Wb_ref[0, 0], preferred_element_type=jnp.float32)
    h = jnp.maximum(h + be_ref[0, 0], 0.0)

    lane = jax.lax.broadcasted_iota(jnp.int32, probs_ref.shape, 1)
    p_col = jnp.sum(
        jnp.where(lane == e, probs_ref[...], 0.0), axis=1, keepdims=True
    )
    contrib = h * p_col

    @pl.when(e == 0)
    def _first():
        out_ref[...] = contrib

    @pl.when(e > 0)
    def _rest():
        out_ref[...] += contrib


def kernel(x, Wg, bg, We, be):
    T, H = x.shape
    E = We.shape[0]
    bg2 = bg.reshape(1, E)
    be3 = be.reshape(E, 1, H)
    We4 = We.reshape(E, 2, H // 2, H)
    return pl.pallas_call(
        _moe_body,
        grid=(E,),
        in_specs=[
            pl.BlockSpec((T, H), lambda e: (0, 0)),
            pl.BlockSpec((H, E), lambda e: (0, 0)),
            pl.BlockSpec((1, E), lambda e: (0, 0)),
            pl.BlockSpec((1, 1, H // 2, H), lambda e: (e, 0, 0, 0)),
            pl.BlockSpec((1, 1, H // 2, H), lambda e: (e, 1, 0, 0)),
            pl.BlockSpec((1, 1, H), lambda e: (e, 0, 0)),
        ],
        out_specs=pl.BlockSpec((T, H), lambda e: (0, 0)),
        out_shape=jax.ShapeDtypeStruct((T, H), jnp.float32),
        scratch_shapes=[
            pltpu.VMEM((T, E), jnp.float32),
        ],
        compiler_params=pltpu.CompilerParams(
            dimension_semantics=("arbitrary",),
        ),
    )(x, Wg, bg2, We4, We4, be3)
